# baseline (speedup 1.0000x reference)
import functools

import jax
import jax.numpy as jnp
from jax.experimental import pallas as pl
from jax.experimental.pallas import tpu as pltpu

_EPS = 1e-12
_C = 8192
_NSETS = 4
_NEG = -jnp.inf


def _topk_kernel(y_ref, x_ref, o_ref, m1, m2, m3, idx, acc, *, r, cols, nc):
    j = pl.program_id(1)
    w = 128 * _NSETS
    nsub = _C // 128
    tail = cols - (nc - 1) * _C
    nsub_tail = pl.cdiv(tail, 128)
    first_masked = tail // 128

    @pl.when(j == 0)
    def _init():
        m1[...] = jnp.full((r, w), _NEG, jnp.float32)
        m2[...] = jnp.full((r, w), _NEG, jnp.float32)
        m3[...] = jnp.full((r, w), _NEG, jnp.float32)
        idx[...] = jnp.zeros((r, w), jnp.int32)
        acc[...] = jnp.zeros((r, w), jnp.float32)

    yb = y_ref[0, 0, :][:, None]
    lane = jax.lax.broadcasted_iota(jnp.int32, (r, 128), 1)

    def sweep(n_sub, masked):
        group = 128 * _NSETS
        n_grp = n_sub // _NSETS
        rem = n_sub % _NSETS
        base = j * _C

        def update(state, k, v, cidx):
            m1v, m2v, m3v, idxv, accv = state
            if masked:
                v = jnp.where(cidx < cols, v, _NEG)
            om1 = m1v[k]
            om2 = m2v[k]
            ge = v >= om1
            idxv = idxv[:k] + (jnp.where(ge, cidx, idxv[k]),) + idxv[k + 1:]
            nm1 = jnp.maximum(om1, v)
            nm2 = jnp.minimum(om1, jnp.maximum(om2, v))
            nm3 = jnp.minimum(om2, jnp.maximum(m3v[k], v))
            m1v = m1v[:k] + (nm1,) + m1v[k + 1:]
            m2v = m2v[:k] + (nm2,) + m2v[k + 1:]
            m3v = m3v[:k] + (nm3,) + m3v[k + 1:]
            accv = (accv[:k] + (accv[k] + jnp.where(cidx == yb, v, 0.0),)
                    + accv[k + 1:])
            return (m1v, m2v, m3v, idxv, accv)

        def body(t, state):
            off = t * group
            for k in range(_NSETS):
                v = x_ref[:, pl.ds(off + k * 128, 128)]
                cidx = lane + (base + off + k * 128)
                state = update(state, k, v, cidx)
            return state

        state = tuple(
            tuple(ref[:, k * 128:(k + 1) * 128] for k in range(_NSETS))
            for ref in (m1, m2, m3, idx, acc))
        state = jax.lax.fori_loop(0, n_grp, body, state)
        for s in range(n_grp * _NSETS, n_sub):
            v = x_ref[:, s * 128:(s + 1) * 128]
            cidx = lane + (base + s * 128)
            state = update(state, s % _NSETS, v, cidx)
        m1v, m2v, m3v, idxv, accv = state
        m1[...] = jnp.concatenate(m1v, axis=1)
        m2[...] = jnp.concatenate(m2v, axis=1)
        m3[...] = jnp.concatenate(m3v, axis=1)
        idx[...] = jnp.concatenate(idxv, axis=1)
        acc[...] = jnp.concatenate(accv, axis=1)

    @pl.when(j < nc - 1)
    def _main():
        sweep(nsub, False)

    @pl.when(j == nc - 1)
    def _tail():
        sweep(nsub_tail, True)

        lanes = jax.lax.broadcasted_iota(jnp.int32, (r, w), 1)
        a1 = m1[...]
        big1 = jnp.max(a1, axis=1, keepdims=True)
        idxmax = jnp.max(jnp.where(a1 == big1, idx[...], -1), axis=1,
                         keepdims=True)
        l1 = jnp.max(jnp.where(a1 == big1, lanes, -1), axis=1, keepdims=True)
        a2 = jnp.where(lanes == l1, m2[...], a1)
        big2 = jnp.max(a2, axis=1, keepdims=True)
        l2 = jnp.max(jnp.where(a2 == big2, lanes, -1), axis=1, keepdims=True)
        a3 = jnp.where(lanes == l2, jnp.where(l1 == l2, m3[...], m2[...]), a2)
        big3 = jnp.max(a3, axis=1, keepdims=True)
        xy = jnp.sum(acc[...], axis=1, keepdims=True)
        ind = idxmax == yb
        num = xy - jnp.where(ind, big2, big1)
        den = big1 - big3 + _EPS
        res = -num / den
        o_ref[0, 0, :] = res[:, 0]


def kernel(x, y):
    rows, cols = x.shape
    r = 8 if rows % 8 == 0 else rows
    nr = rows // r
    nc = pl.cdiv(cols, _C)
    y32 = y.astype(jnp.int32).reshape(nr, 1, r)

    body = functools.partial(_topk_kernel, r=r, cols=cols, nc=nc)
    out = pl.pallas_call(
        body,
        grid=(nr, nc),
        in_specs=[
            pl.BlockSpec((1, 1, r), lambda i, j: (i, 0, 0)),
            pl.BlockSpec((r, _C), lambda i, j: (i, j)),
        ],
        out_specs=pl.BlockSpec((1, 1, r), lambda i, j: (i, 0, 0)),
        out_shape=jax.ShapeDtypeStruct((nr, 1, r), jnp.float32),
        scratch_shapes=[
            pltpu.VMEM((r, 128 * _NSETS), jnp.float32),
            pltpu.VMEM((r, 128 * _NSETS), jnp.float32),
            pltpu.VMEM((r, 128 * _NSETS), jnp.float32),
            pltpu.VMEM((r, 128 * _NSETS), jnp.int32),
            pltpu.VMEM((r, 128 * _NSETS), jnp.float32),
        ],
        compiler_params=pltpu.CompilerParams(
            dimension_semantics=("arbitrary", "arbitrary")),
    )(y32, x)
    return out.reshape(rows)
